# Initial kernel scaffold; baseline (speedup 1.0000x reference)
import jax
import jax.numpy as jnp
from jax.experimental import pallas as pl


def kernel(x, W, b, centroids):
    raise NotImplementedError("write your pallas kernel here")


# speedup vs baseline: 3.5961x; 3.5961x over previous
import jax
import jax.numpy as jnp
from jax.experimental import pallas as pl

_BV = 4096
_BM = 512


def _quant_body(vt_ref, cen_ref, out_ref):
    v = vt_ref[...]
    cen = cen_ref[...]
    k = cen.shape[0]
    n2 = jnp.sum(v * v, axis=0, keepdims=True)
    inv = 1.0 / (jnp.sqrt(n2) + 1e-8)
    nv = v * inv
    c2 = jnp.sum(cen * cen, axis=1, keepdims=True)
    s = jax.lax.dot_general(cen, nv, (((1,), (0,)), ((), ())),
                            preferred_element_type=jnp.float32)
    e = c2 - 2.0 * s
    m = jnp.min(e, axis=0, keepdims=True)
    iota = jax.lax.broadcasted_iota(jnp.int32, e.shape, 0)
    idx = jnp.min(jnp.where(e <= m, iota, k), axis=0, keepdims=True)
    onehot = (iota == idx).astype(jnp.float32)
    assigned = jax.lax.dot_general(
        cen, onehot, (((0,), (0,)), ((), ())),
        preferred_element_type=jnp.float32,
        precision=jax.lax.Precision.HIGHEST)
    num = jnp.sum(v * assigned, axis=0, keepdims=True)
    den = jnp.sum(assigned * assigned, axis=0, keepdims=True) + 1e-8
    out_ref[...] = assigned * (num / den)


def _mm_body(x_ref, wt_ref, b_ref, o_ref):
    o_ref[...] = jnp.dot(x_ref[...], wt_ref[...],
                         preferred_element_type=jnp.float32) + b_ref[...]


def kernel(x, W, b, centroids):
    dout, din = W.shape
    vs = centroids.shape[1]
    kk = centroids.shape[0]
    nv = (dout * din) // vs
    vt = W.reshape(-1, vs).T

    qt = pl.pallas_call(
        _quant_body,
        grid=(nv // _BV,),
        in_specs=[
            pl.BlockSpec((vs, _BV), lambda i: (0, i)),
            pl.BlockSpec((kk, vs), lambda i: (0, 0)),
        ],
        out_specs=pl.BlockSpec((vs, _BV), lambda i: (0, i)),
        out_shape=jax.ShapeDtypeStruct((vs, nv), jnp.float32),
    )(vt, centroids)

    wqt = qt.reshape(vs, dout, din // vs).transpose(2, 0, 1).reshape(din, dout)

    xm = x.reshape(-1, din)
    y = pl.pallas_call(
        _mm_body,
        grid=(xm.shape[0] // _BM,),
        in_specs=[
            pl.BlockSpec((_BM, din), lambda i: (i, 0)),
            pl.BlockSpec((din, dout), lambda i: (0, 0)),
            pl.BlockSpec((1, dout), lambda i: (0, 0)),
        ],
        out_specs=pl.BlockSpec((_BM, dout), lambda i: (i, 0)),
        out_shape=jax.ShapeDtypeStruct((xm.shape[0], dout), jnp.float32),
    )(xm, wqt, b.reshape(1, dout))
    return y.reshape(x.shape)


# speedup vs baseline: 7.6477x; 2.1267x over previous
import jax
import jax.numpy as jnp
from jax.experimental import pallas as pl

_BR = 128
_BM = 512


def _quant_body(w_ref, cen_ref, cent_ref, out_ref):
    w = w_ref[...]
    cen = cen_ref[...]
    cent = cent_ref[...]
    kk = cen.shape[0]
    br = w.shape[0]
    vs = cen.shape[1]
    nb = w.shape[1] // vs
    bv = br * nb
    v = w.reshape(br, nb, vs).transpose(2, 0, 1).reshape(vs, bv)
    n2 = jnp.sum(v * v, axis=0, keepdims=True)
    inv = 1.0 / (jnp.sqrt(n2) + 1e-8)
    nv = v * inv
    c2 = jnp.sum(cen * cen, axis=1, keepdims=True)
    s = jax.lax.dot_general(cen, nv, (((1,), (0,)), ((), ())),
                            preferred_element_type=jnp.float32)
    e = c2 - 2.0 * s
    m = jnp.min(e, axis=0, keepdims=True)
    iota = jax.lax.broadcasted_iota(jnp.int32, e.shape, 0)
    idx = jnp.min(jnp.where(e <= m, iota, kk), axis=0, keepdims=True)
    half = 128
    idm = jnp.where(idx < half, idx, idx - half)
    idxb = jnp.broadcast_to(idm, (vs, bv))
    a_lo = jnp.take_along_axis(cent[:, :half], idxb, axis=1)
    a_hi = jnp.take_along_axis(cent[:, half:], idxb, axis=1)
    assigned = jnp.where(idx < half, a_lo, a_hi)
    num = jnp.sum(v * assigned, axis=0, keepdims=True)
    den = jnp.sum(assigned * assigned, axis=0, keepdims=True) + 1e-8
    q = assigned * (num / den)
    out_ref[...] = q.reshape(vs, br, nb).transpose(2, 0, 1).reshape(w.shape[1], br)


def _mm_body(x_ref, wt_ref, b_ref, o_ref):
    o_ref[...] = jnp.dot(x_ref[...], wt_ref[...],
                         preferred_element_type=jnp.float32) + b_ref[...]


def kernel(x, W, b, centroids):
    dout, din = W.shape
    vs = centroids.shape[1]
    kk = centroids.shape[0]

    wqt = pl.pallas_call(
        _quant_body,
        grid=(dout // _BR,),
        in_specs=[
            pl.BlockSpec((_BR, din), lambda i: (i, 0)),
            pl.BlockSpec((kk, vs), lambda i: (0, 0)),
            pl.BlockSpec((vs, kk), lambda i: (0, 0)),
        ],
        out_specs=pl.BlockSpec((din, _BR), lambda i: (0, i)),
        out_shape=jax.ShapeDtypeStruct((din, dout), jnp.float32),
    )(W, centroids, centroids.T)

    xm = x.reshape(-1, din)
    y = pl.pallas_call(
        _mm_body,
        grid=(xm.shape[0] // _BM,),
        in_specs=[
            pl.BlockSpec((_BM, din), lambda i: (i, 0)),
            pl.BlockSpec((din, dout), lambda i: (0, 0)),
            pl.BlockSpec((1, dout), lambda i: (0, 0)),
        ],
        out_specs=pl.BlockSpec((_BM, dout), lambda i: (i, 0)),
        out_shape=jax.ShapeDtypeStruct((xm.shape[0], dout), jnp.float32),
    )(xm, wqt, b.reshape(1, dout))
    return y.reshape(x.shape)


# speedup vs baseline: 9.1267x; 1.1934x over previous
import jax
import jax.numpy as jnp
from jax.experimental import pallas as pl

_BR = 128
_BM = 512


def _quant_body(w_ref, cen_ref, cent_ref, out_ref):
    w = w_ref[...]
    cen = cen_ref[...]
    cent = cent_ref[...]
    kk = cen.shape[0]
    br = w.shape[0]
    vs = cen.shape[1]
    nb = w.shape[1] // vs
    bv = br * nb
    v = w.reshape(br, nb, vs).transpose(2, 0, 1).reshape(vs, bv)
    n2 = jnp.sum(v * v, axis=0, keepdims=True)
    inv = 1.0 / (jnp.sqrt(n2) + 1e-8)
    nv = v * inv
    c2 = jnp.sum(cen * cen, axis=1, keepdims=True)
    s = jax.lax.dot_general(-2.0 * cen, nv, (((1,), (0,)), ((), ())),
                            preferred_element_type=jnp.float32)
    e = c2 + s
    idx = jnp.argmin(e, axis=0).reshape(1, bv).astype(jnp.int32)
    half = 128
    idm = jnp.where(idx < half, idx, idx - half)
    idxb = jnp.broadcast_to(idm, (vs, bv))
    a_lo = jnp.take_along_axis(cent[:, :half], idxb, axis=1)
    a_hi = jnp.take_along_axis(cent[:, half:], idxb, axis=1)
    assigned = jnp.where(idx < half, a_lo, a_hi)
    num = jnp.sum(v * assigned, axis=0, keepdims=True)
    den = jnp.sum(assigned * assigned, axis=0, keepdims=True) + 1e-8
    q = assigned * (num / den)
    out_ref[...] = q.reshape(vs, br, nb).transpose(2, 0, 1).reshape(w.shape[1], br)


def _mm_body(x_ref, wt_ref, b_ref, o_ref):
    o_ref[...] = jnp.dot(x_ref[...], wt_ref[...],
                         preferred_element_type=jnp.float32) + b_ref[...]


def kernel(x, W, b, centroids):
    dout, din = W.shape
    vs = centroids.shape[1]
    kk = centroids.shape[0]

    wqt = pl.pallas_call(
        _quant_body,
        grid=(dout // _BR,),
        in_specs=[
            pl.BlockSpec((_BR, din), lambda i: (i, 0)),
            pl.BlockSpec((kk, vs), lambda i: (0, 0)),
            pl.BlockSpec((vs, kk), lambda i: (0, 0)),
        ],
        out_specs=pl.BlockSpec((din, _BR), lambda i: (0, i)),
        out_shape=jax.ShapeDtypeStruct((din, dout), jnp.float32),
    )(W, centroids, centroids.T)

    xm = x.reshape(-1, din)
    y = pl.pallas_call(
        _mm_body,
        grid=(xm.shape[0] // _BM,),
        in_specs=[
            pl.BlockSpec((_BM, din), lambda i: (i, 0)),
            pl.BlockSpec((din, dout), lambda i: (0, 0)),
            pl.BlockSpec((1, dout), lambda i: (0, 0)),
        ],
        out_specs=pl.BlockSpec((_BM, dout), lambda i: (i, 0)),
        out_shape=jax.ShapeDtypeStruct((xm.shape[0], dout), jnp.float32),
    )(xm, wqt, b.reshape(1, dout))
    return y.reshape(x.shape)


# speedup vs baseline: 9.3414x; 1.0235x over previous
import jax
import jax.numpy as jnp
from jax.experimental import pallas as pl

_BR = 128
_BM = 1024


def _quant_body(w_ref, cen_ref, cent_ref, out_ref):
    w = w_ref[...]
    cen = cen_ref[...]
    cent = cent_ref[...]
    kk = cen.shape[0]
    br = w.shape[0]
    vs = cen.shape[1]
    nb = w.shape[1] // vs
    bv = br * nb
    v = w.reshape(br, nb, vs).transpose(2, 0, 1).reshape(vs, bv)
    n2 = jnp.sum(v * v, axis=0, keepdims=True)
    inv = 1.0 / (jnp.sqrt(n2) + 1e-8)
    nv = v * inv
    c2 = jnp.sum(cen * cen, axis=1, keepdims=True)
    s = jax.lax.dot_general(-2.0 * cen, nv, (((1,), (0,)), ((), ())),
                            preferred_element_type=jnp.float32)
    e = c2 + s
    idx = jnp.argmin(e, axis=0).reshape(1, bv).astype(jnp.int32)
    half = 128
    idm = jnp.where(idx < half, idx, idx - half)
    idxb = jnp.broadcast_to(idm, (vs, bv))
    a_lo = jnp.take_along_axis(cent[:, :half], idxb, axis=1)
    a_hi = jnp.take_along_axis(cent[:, half:], idxb, axis=1)
    assigned = jnp.where(idx < half, a_lo, a_hi)
    num = jnp.sum(v * assigned, axis=0, keepdims=True)
    den = jnp.sum(assigned * assigned, axis=0, keepdims=True) + 1e-8
    q = assigned * (num / den)
    out_ref[...] = q.reshape(vs, br, nb).transpose(2, 0, 1).reshape(w.shape[1], br)


def _mm_body(x_ref, wt_ref, b_ref, o_ref):
    o_ref[...] = jnp.dot(x_ref[...], wt_ref[...],
                         preferred_element_type=jnp.float32) + b_ref[...]


def kernel(x, W, b, centroids):
    dout, din = W.shape
    vs = centroids.shape[1]
    kk = centroids.shape[0]

    wqt = pl.pallas_call(
        _quant_body,
        grid=(dout // _BR,),
        in_specs=[
            pl.BlockSpec((_BR, din), lambda i: (i, 0)),
            pl.BlockSpec((kk, vs), lambda i: (0, 0)),
            pl.BlockSpec((vs, kk), lambda i: (0, 0)),
        ],
        out_specs=pl.BlockSpec((din, _BR), lambda i: (0, i)),
        out_shape=jax.ShapeDtypeStruct((din, dout), jnp.float32),
    )(W, centroids, centroids.T)

    xm = x.reshape(-1, din)
    y = pl.pallas_call(
        _mm_body,
        grid=(xm.shape[0] // _BM,),
        in_specs=[
            pl.BlockSpec((_BM, din), lambda i: (i, 0)),
            pl.BlockSpec((din, dout), lambda i: (0, 0)),
            pl.BlockSpec((1, dout), lambda i: (0, 0)),
        ],
        out_specs=pl.BlockSpec((_BM, dout), lambda i: (i, 0)),
        out_shape=jax.ShapeDtypeStruct((xm.shape[0], dout), jnp.float32),
    )(xm, wqt, b.reshape(1, dout))
    return y.reshape(x.shape)


# speedup vs baseline: 13.3935x; 1.4338x over previous
import jax
import jax.numpy as jnp
from jax.experimental import pallas as pl

_BR = 128
_BM = 1024


def _quant_body(w_ref, cen_ref, cent_ref, out_ref):
    w = w_ref[...]
    cen = cen_ref[...]
    cent = cent_ref[...]
    kk = cen.shape[0]
    br = w.shape[0]
    vs = cen.shape[1]
    nb = w.shape[1] // vs
    bv = br * nb
    wt = w.T
    v = wt.reshape(nb, vs, br).transpose(1, 0, 2).reshape(vs, bv)
    n2 = jnp.sum(v * v, axis=0, keepdims=True)
    inv = 1.0 / (jnp.sqrt(n2) + 1e-8)
    nv = v * inv
    c2 = jnp.sum(cen * cen, axis=1, keepdims=True)
    s = jax.lax.dot_general(-2.0 * cen, nv, (((1,), (0,)), ((), ())),
                            preferred_element_type=jnp.float32)
    e = c2 + s
    idx = jnp.argmin(e, axis=0).reshape(1, bv).astype(jnp.int32)
    half = 128
    idm = jnp.where(idx < half, idx, idx - half)
    idxb = jnp.broadcast_to(idm, (vs, bv))
    a_lo = jnp.take_along_axis(cent[:, :half], idxb, axis=1)
    a_hi = jnp.take_along_axis(cent[:, half:], idxb, axis=1)
    assigned = jnp.where(idx < half, a_lo, a_hi)
    num = jnp.sum(v * assigned, axis=0, keepdims=True)
    den = jnp.sum(assigned * assigned, axis=0, keepdims=True) + 1e-8
    q = assigned * (num / den)
    out_ref[...] = q.reshape(vs, nb, br).transpose(1, 0, 2).reshape(w.shape[1], br)


def _mm_body(x_ref, wt_ref, b_ref, o_ref):
    o_ref[...] = jnp.dot(x_ref[...], wt_ref[...],
                         preferred_element_type=jnp.float32) + b_ref[...]


def kernel(x, W, b, centroids):
    dout, din = W.shape
    vs = centroids.shape[1]
    kk = centroids.shape[0]

    wqt = pl.pallas_call(
        _quant_body,
        grid=(dout // _BR,),
        in_specs=[
            pl.BlockSpec((_BR, din), lambda i: (i, 0)),
            pl.BlockSpec((kk, vs), lambda i: (0, 0)),
            pl.BlockSpec((vs, kk), lambda i: (0, 0)),
        ],
        out_specs=pl.BlockSpec((din, _BR), lambda i: (0, i)),
        out_shape=jax.ShapeDtypeStruct((din, dout), jnp.float32),
    )(W, centroids, centroids.T)

    xm = x.reshape(-1, din)
    y = pl.pallas_call(
        _mm_body,
        grid=(xm.shape[0] // _BM,),
        in_specs=[
            pl.BlockSpec((_BM, din), lambda i: (i, 0)),
            pl.BlockSpec((din, dout), lambda i: (0, 0)),
            pl.BlockSpec((1, dout), lambda i: (0, 0)),
        ],
        out_specs=pl.BlockSpec((_BM, dout), lambda i: (i, 0)),
        out_shape=jax.ShapeDtypeStruct((xm.shape[0], dout), jnp.float32),
    )(xm, wqt, b.reshape(1, dout))
    return y.reshape(x.shape)
